# baseline (speedup 1.0000x reference)
import jax
import jax.numpy as jnp
from jax.experimental import pallas as pl
from jax.experimental.pallas import tpu as pltpu


def _body(pos_ref, prev_ref, cur_ref, out_ref, sem):
    del prev_ref
    p = pos_ref[0]
    cp = pltpu.make_async_copy(
        cur_ref, out_ref.at[:, :, pl.ds(p, 1), :], sem)
    cp.start()
    cp.wait()


def kernel(prev, cur, dim, idx):
    pos = (idx - (dim - 1)).astype(jnp.int32)
    one = (idx[0] > -(1 << 30)).astype(prev.dtype)
    tmp = prev * one
    out = pl.pallas_call(
        _body,
        grid_spec=pltpu.PrefetchScalarGridSpec(
            num_scalar_prefetch=1,
            grid=(1,),
            in_specs=[
                pl.BlockSpec(memory_space=pl.ANY),
                pl.BlockSpec(memory_space=pl.ANY),
            ],
            out_specs=pl.BlockSpec(memory_space=pl.ANY),
            scratch_shapes=[pltpu.SemaphoreType.DMA],
        ),
        out_shape=jax.ShapeDtypeStruct(prev.shape, prev.dtype),
        input_output_aliases={1: 0},
    )(pos, tmp, cur)
    return out
